# baseline (speedup 1.0000x reference)
import functools
import math

import jax
import jax.numpy as jnp
from jax.experimental import pallas as pl

_L = 64
_K = 128
_B = 4096
_BB = 128
_HALF_LOG_2PI = 0.5 * math.log(2.0 * math.pi)


def _tc_body(z_ref, mu_ref, lv_ref, w_ref, out_ref):
    z = z_ref[...]
    mu = mu_ref[...]
    lv = lv_ref[...]
    w = w_ref[...]
    wmax = jnp.max(w)
    lw = w - (wmax + jnp.log(jnp.sum(jnp.exp(w - wmax))))
    prec = jnp.exp(-lv)
    diff = z[None, :, :] - mu[:, None, :]
    t = (-_HALF_LOG_2PI - 0.5 * lv[:, None, :]
         - 0.5 * prec[:, None, :] * diff * diff
         + lw[0][:, None, None])
    m = jnp.max(t, axis=0)
    s = jnp.sum(jnp.exp(t - m[None, :, :]), axis=0)
    out_ref[...] = m + jnp.log(s)


@jax.jit
def kernel(z, means, logvars, w):
    w2 = w.reshape(1, _K)
    out = pl.pallas_call(
        _tc_body,
        grid=(_B // _BB,),
        in_specs=[
            pl.BlockSpec((_BB, _L), lambda i: (i, 0)),
            pl.BlockSpec((_K, _L), lambda i: (0, 0)),
            pl.BlockSpec((_K, _L), lambda i: (0, 0)),
            pl.BlockSpec((1, _K), lambda i: (0, 0)),
        ],
        out_specs=pl.BlockSpec((_BB, _L), lambda i: (i, 0)),
        out_shape=jax.ShapeDtypeStruct((_B, _L), jnp.float32),
    )(z, means, logvars, w2)
    return out.T


# speedup vs baseline: 2.3382x; 2.3382x over previous
import math

import jax
import jax.numpy as jnp
from jax.experimental import pallas as pl

_L = 64
_K = 128
_B = 4096
_R = _B // 2
_W = 2 * _L
_BB = 64
_HALF_LOG_2PI = 0.5 * math.log(2.0 * math.pi)


def _tc_body(z_ref, mu_ref, lv_ref, w_ref, out_ref):
    z = z_ref[...]
    mu = mu_ref[...]
    lv = lv_ref[...]
    w = w_ref[...]
    wmax = jnp.max(w)
    lw = w - (wmax + jnp.log(jnp.sum(jnp.exp(w - wmax))))
    nhalfprec = -0.5 * jnp.exp(-lv)
    base = (-_HALF_LOG_2PI - 0.5 * lv) + lw[0][:, None]
    diff = z[None, :, :] - mu[:, None, :]
    t = base[:, None, :] + nhalfprec[:, None, :] * (diff * diff)
    m = jnp.max(t, axis=0)
    s = jnp.sum(jnp.exp(t - m[None, :, :]), axis=0)
    out_ref[...] = m + jnp.log(s)


@jax.jit
def kernel(z, means, logvars, w):
    z2d = z.reshape(_R, _W)
    mu2 = jnp.tile(means, (1, 2))
    lv2 = jnp.tile(logvars, (1, 2))
    w2 = w.reshape(1, _K)
    out = pl.pallas_call(
        _tc_body,
        grid=(_R // _BB,),
        in_specs=[
            pl.BlockSpec((_BB, _W), lambda i: (i, 0)),
            pl.BlockSpec((_K, _W), lambda i: (0, 0)),
            pl.BlockSpec((_K, _W), lambda i: (0, 0)),
            pl.BlockSpec((1, _K), lambda i: (0, 0)),
        ],
        out_specs=pl.BlockSpec((_BB, _W), lambda i: (i, 0)),
        out_shape=jax.ShapeDtypeStruct((_R, _W), jnp.float32),
    )(z2d, mu2, lv2, w2)
    return out.reshape(_B, _L).T
